# baseline (speedup 1.0000x reference)
import jax
import jax.numpy as jnp
from jax.experimental import pallas as pl
from jax.experimental.pallas import tpu as pltpu

B, H, S, D, Q = 16, 8, 2048, 128, 1
HC = 4
NBUF = 8


def _body(pos_ref, k_hbm, v_hbm, lk_ref, lv_ref, ok_hbm, ov_hbm, *scratch):
    bufs = scratch[:NBUF]
    isems = scratch[NBUF:2 * NBUF]
    osems = scratch[2 * NBUF:]

    chunks = [(k_hbm, ok_hbm, lk_ref, b, h0)
              for b in range(B) for h0 in range(0, H, HC)]
    chunks += [(v_hbm, ov_hbm, lv_ref, b, h0)
               for b in range(B) for h0 in range(0, H, HC)]
    n = len(chunks)
    prefetch = NBUF - 1

    def start_in(t):
        src, _, _, b, h0 = chunks[t]
        cp = pltpu.make_async_copy(
            src.at[b, h0:h0 + HC], bufs[t % NBUF], isems[t % NBUF])
        cp.start()
        return cp

    ins = [None] * NBUF
    outs = [None] * NBUF
    for t in range(min(prefetch, n)):
        ins[t % NBUF] = start_in(t)
    for t in range(n):
        nb = t % NBUF
        _, dst, lat, b, h0 = chunks[t]
        ins[nb].wait()
        local = pos_ref[b]
        bufs[nb][:, pl.ds(local, 1), :] = lat[b, h0:h0 + HC]
        cp_out = pltpu.make_async_copy(bufs[nb], dst.at[b, h0:h0 + HC],
                                       osems[nb])
        cp_out.start()
        outs[nb] = cp_out
        tp = t + prefetch
        if tp < n:
            bp = tp % NBUF
            if outs[bp] is not None:
                outs[bp].wait()
                outs[bp] = None
            ins[bp] = start_in(tp)
    for cp in outs:
        if cp is not None:
            cp.wait()


def kernel(k_cache, v_cache, latest_k, latest_v, position_ids):
    pos = position_ids.reshape(B).astype(jnp.int32)
    out_shape = [
        jax.ShapeDtypeStruct((B, H, S, D), k_cache.dtype),
        jax.ShapeDtypeStruct((B, H, S, D), v_cache.dtype),
    ]
    scratch_shapes = (
        [pltpu.VMEM((HC, S, D), jnp.float32)] * NBUF
        + [pltpu.SemaphoreType.DMA] * (2 * NBUF)
    )
    k_new, v_new = pl.pallas_call(
        _body,
        grid=(),
        in_specs=[
            pl.BlockSpec(memory_space=pltpu.SMEM),
            pl.BlockSpec(memory_space=pl.ANY),
            pl.BlockSpec(memory_space=pl.ANY),
            pl.BlockSpec(memory_space=pltpu.VMEM),
            pl.BlockSpec(memory_space=pltpu.VMEM),
        ],
        out_specs=[
            pl.BlockSpec(memory_space=pl.ANY),
            pl.BlockSpec(memory_space=pl.ANY),
        ],
        out_shape=out_shape,
        scratch_shapes=scratch_shapes,
    )(pos, k_cache, v_cache, latest_k, latest_v)
    return (k_new, v_new)
